# baseline (speedup 1.0000x reference)
import functools

import jax
import jax.numpy as jnp
from jax import lax
from jax.experimental import pallas as pl
from jax.experimental.pallas import tpu as pltpu
from jax.experimental.pallas import tpu_sc as plsc

_D = 64
_DP = 128
_H = 50
_HP = 56
_BATCH = 16384
_NW = 32
_BPW = _BATCH // _NW
_CB = 4
_NCHK = _BPW // _CB
_NBUF = 2
_RPW = _BPW * _HP


def _gather_sc(idx_grp, table_p):
    mesh = plsc.VectorSubcoreMesh(core_axis_name="c", subcore_axis_name="s")

    @functools.partial(
        pl.kernel,
        mesh=mesh,
        out_type=jax.ShapeDtypeStruct((_BATCH, _HP, _DP), jnp.float32),
        scratch_types=[
            pltpu.VMEM((_RPW,), jnp.int32),
            pltpu.VMEM((_NBUF, _CB, _HP, _DP), jnp.float32),
            pltpu.SemaphoreType.DMA,
            pltpu.SemaphoreType.DMA,
        ],
    )
    def k(idx_hbm, table_hbm, out_hbm, idx_v, rows_v, gsem0, gsem1):
        gsems = (gsem0, gsem1)
        wid = lax.axis_index("s") * 2 + lax.axis_index("c")
        base_b = wid * _BPW
        pltpu.sync_copy(idx_hbm.at[wid], idx_v)

        def start_gather(g, nb):
            for j in range(_CB):
                pltpu.async_copy(
                    table_hbm.at[idx_v.at[pl.ds((g * _CB + j) * _HP, _HP)]],
                    rows_v.at[nb, j],
                    gsems[nb],
                )

        def wait_gather(g, nb):
            for j in range(_CB):
                pltpu.make_async_copy(
                    table_hbm.at[idx_v.at[pl.ds((g * _CB + j) * _HP, _HP)]],
                    rows_v.at[nb, j],
                    gsems[nb],
                ).wait()

        for nb in range(_NBUF):
            start_gather(nb, nb)

        def body(t, carry):
            for nb in range(_NBUF):
                g = t * _NBUF + nb
                wait_gather(g, nb)
                pltpu.sync_copy(
                    rows_v.at[nb],
                    out_hbm.at[pl.ds(base_b + g * _CB, _CB)],
                )

                @pl.when(g + _NBUF < _NCHK)
                def _():
                    start_gather(g + _NBUF, nb)

            return carry

        lax.fori_loop(0, _NCHK // _NBUF, body, 0)

    return k(idx_grp, table_p)


def kernel(input, table):
    idx56 = jnp.pad(input.astype(jnp.int32), ((0, 0), (0, _HP - _H)))
    idx_grp = idx56.reshape(_NW, _RPW)
    table_p = jnp.pad(table, ((0, 0), (0, _DP - _D)))
    out_p = _gather_sc(idx_grp, table_p)
    return out_p[:, :_H, :_D]
